# baseline (speedup 1.0000x reference)
import functools

import jax
import jax.numpy as jnp
from jax import lax
from jax.experimental import pallas as pl
from jax.experimental.pallas import tpu as pltpu
from jax.experimental.pallas import tpu_sc as plsc

NN = 10000
EE = 320000
HH = 128
NPAD = 10240
NCORE = 2
NSUB = 16
NWORK = NCORE * NSUB
EPW = EE // NWORK
CHUNK = 80
NCHUNK = EPW // CHUNK
RPT = NPAD // NSUB
RBLK = 1024
NBLK = NPAD // RBLK

_mesh = plsc.VectorSubcoreMesh(core_axis_name="c", subcore_axis_name="s")


@functools.partial(
    pl.kernel,
    out_type=jax.ShapeDtypeStruct((NCORE, NPAD), jnp.float32),
    mesh=_mesh,
    scratch_types=[
        pltpu.VMEM((NCHUNK, CHUNK), jnp.int32),
        pltpu.VMEM((EPW,), jnp.float32),
        pltpu.VMEM_SHARED((NPAD,), jnp.float32),
        pltpu.SemaphoreType.DMA,
    ],
)
def _sc_deg(col_hbm, ea_hbm, zrow_hbm, deg_hbm, col_v, w_v, acc_sh, dsem):
    cid = lax.axis_index("c")
    sid = lax.axis_index("s")
    wid = cid * NSUB + sid
    pltpu.sync_copy(col_hbm.at[wid], col_v)
    pltpu.sync_copy(ea_hbm.at[wid], w_v)

    def _abs_body(i, carry):
        w_v[pl.ds(i * 16, 16)] = jnp.abs(w_v[pl.ds(i * 16, 16)])
        return carry

    lax.fori_loop(0, EPW // 16, _abs_body, 0)
    pltpu.sync_copy(zrow_hbm, acc_sh.at[pl.ds(sid * RPT, RPT)])
    plsc.subcore_barrier()

    fire = 25

    def _round_body(r, carry):
        for j in range(fire):
            pltpu.async_copy(
                w_v.at[pl.ds((r * fire + j) * CHUNK, CHUNK)],
                acc_sh.at[col_v.at[r * fire + j]],
                dsem,
                add=True,
            )
        for j in range(fire):
            pltpu.make_async_copy(
                w_v.at[pl.ds(j * CHUNK, CHUNK)],
                acc_sh.at[col_v.at[j]],
                dsem,
            ).wait()
        return carry

    lax.fori_loop(0, NCHUNK // fire, _round_body, 0)
    plsc.subcore_barrier()
    pltpu.sync_copy(acc_sh.at[pl.ds(sid * RPT, RPT)],
                    deg_hbm.at[cid, pl.ds(sid * RPT, RPT)])


@functools.partial(
    pl.kernel,
    out_type=jax.ShapeDtypeStruct((NCORE, NPAD, HH), jnp.float32),
    mesh=_mesh,
    scratch_types=[
        pltpu.VMEM((EPW,), jnp.int32),
        pltpu.VMEM((CHUNK, HH), jnp.float32),
        pltpu.VMEM((CHUNK, HH), jnp.float32),
        pltpu.VMEM((CHUNK, HH), jnp.float32),
        pltpu.VMEM((CHUNK,), jnp.float32),
        pltpu.VMEM((CHUNK,), jnp.float32),
        pltpu.VMEM((CHUNK,), jnp.float32),
        pltpu.VMEM((CHUNK,), jnp.int32),
        pltpu.VMEM((CHUNK,), jnp.int32),
        pltpu.VMEM((CHUNK,), jnp.int32),
        pltpu.VMEM_SHARED((NPAD, HH), jnp.float32),
        pltpu.SemaphoreType.DMA,
        pltpu.SemaphoreType.DMA,
        pltpu.SemaphoreType.DMA,
        pltpu.SemaphoreType.DMA,
        pltpu.SemaphoreType.DMA,
        pltpu.SemaphoreType.DMA,
    ],
)
def _sc_agg(row_hbm, col_hbm, ea_hbm, g_hbm, ztile_hbm, acc_hbm,
            row_v, m0, m1, m2, e0, e1, e2, c0, c1, c2, acc_sh,
            g0, g1, g2, s0, s1, s2):
    cid = lax.axis_index("c")
    sid = lax.axis_index("s")
    wid = cid * NSUB + sid
    pltpu.sync_copy(row_hbm.at[wid], row_v)
    pltpu.sync_copy(ztile_hbm, acc_sh.at[pl.ds(sid * RPT, RPT)])
    plsc.subcore_barrier()

    msg = (m0, m1, m2)
    eab = (e0, e1, e2)
    colb = (c0, c1, c2)
    gsem = (g0, g1, g2)
    ssem = (s0, s1, s2)

    def _gather_start(k, b):
        pltpu.async_copy(g_hbm.at[row_v.at[pl.ds(k * CHUNK, CHUNK)]],
                         msg[b], gsem[b])
        pltpu.async_copy(ea_hbm.at[pl.ds(wid * EPW + k * CHUNK, CHUNK)],
                         eab[b], gsem[b])
        pltpu.async_copy(col_hbm.at[pl.ds(wid * EPW + k * CHUNK, CHUNK)],
                         colb[b], gsem[b])

    def _gather_wait(b):
        pltpu.make_async_copy(g_hbm.at[row_v.at[pl.ds(0, CHUNK)]],
                              msg[b], gsem[b]).wait()
        pltpu.make_async_copy(ea_hbm.at[pl.ds(0, CHUNK)],
                              eab[b], gsem[b]).wait()
        pltpu.make_async_copy(col_hbm.at[pl.ds(0, CHUNK)],
                              colb[b], gsem[b]).wait()

    def _scatter_start(b):
        pltpu.async_copy(msg[b], acc_sh.at[colb[b]], ssem[b], add=True)

    def _scatter_wait(b):
        pltpu.make_async_copy(msg[b], acc_sh.at[colb[b]], ssem[b]).wait()

    def _scale(b):
        buf = msg[b]
        for j in range(CHUNK // 16):
            wvec = jnp.abs(eab[b][pl.ds(j * 16, 16)])
            for l in range(16):
                e = j * 16 + l
                wv = jnp.full((16,), wvec[l], jnp.float32)
                for cg in range(HH // 16):
                    buf[e, pl.ds(cg * 16, 16)] = (
                        buf[e, pl.ds(cg * 16, 16)] * wv)

    def _phase(k, b):
        _gather_wait(b)
        b1 = (b + 1) % 3

        @pl.when(k + 1 < NCHUNK)
        def _prefetch():
            @pl.when(k >= 2)
            def _drain():
                _scatter_wait(b1)

            _gather_start(k + 1, b1)

        _scale(b)
        _scatter_start(b)

    _gather_start(0, 0)

    def _chunk_body(k, carry):
        for b in range(3):
            @pl.when(k % 3 == b)
            def _run(b=b):
                _phase(k, b)

        return carry

    lax.fori_loop(0, NCHUNK, _chunk_body, 0)
    _scatter_wait(0)
    _scatter_wait(1)
    _scatter_wait(2)
    plsc.subcore_barrier()
    pltpu.sync_copy(acc_sh.at[pl.ds(sid * RPT, RPT)],
                    acc_hbm.at[cid, pl.ds(sid * RPT, RPT)])


def _leaky(x):
    return jnp.where(x > 0, x, 0.2 * x)


def _dinv_from(dp_ref):
    deg = dp_ref[0] + dp_ref[1] + 1.0
    return jnp.where(deg > 0, lax.rsqrt(deg), 0.0)


def _prep_body(dp_ref, x_ref, w_ref, h_ref, g_ref):
    dinv = _dinv_from(dp_ref)
    h = jnp.dot(x_ref[...], w_ref[...], preferred_element_type=jnp.float32)
    h_ref[...] = h
    g_ref[...] = h * dinv


def _block_layer_body(dp_ref, accp_ref, h_ref, b_ref, lw_ref, lb_ref,
                      t_ref, stats_ref, *, mask_rows):
    i = pl.program_id(0)
    dinv = _dinv_from(dp_ref)
    acc = accp_ref[0] + accp_ref[1]
    zp = acc * dinv + h_ref[...] * (dinv * dinv) + b_ref[...]
    t = _leaky(jnp.dot(zp, lw_ref[...], preferred_element_type=jnp.float32)
               + lb_ref[...])
    if mask_rows:
        rows = lax.broadcasted_iota(jnp.int32, (RBLK, HH), 0) + i * RBLK
        t = jnp.where(rows < NN, t, 0.0)
    if t_ref is not None:
        t_ref[...] = t
    s1 = jnp.sum(t, axis=0, keepdims=True)
    s2 = jnp.sum(t * t, axis=0, keepdims=True)
    st = jnp.concatenate([s1, s2], axis=0)

    @pl.when(i == 0)
    def _init():
        stats_ref[...] = st

    @pl.when(i > 0)
    def _accum():
        stats_ref[...] = stats_ref[...] + st


def _layer0_body(dp_ref, accp_ref, h_ref, b_ref, lw_ref, lb_ref,
                 t_ref, stats_ref):
    _block_layer_body(dp_ref, accp_ref, h_ref, b_ref, lw_ref, lb_ref,
                      t_ref, stats_ref, mask_rows=True)


def _tail_body(dp_ref, accp_ref, h_ref, b_ref, lw_ref, lb_ref,
               bng_ref, bnb_ref, f1w_ref, f1b_ref, f2w_ref, f2b_ref,
               f3w_ref, f3b_ref, out_ref, stats_vmem):
    i = pl.program_id(0)
    dinv = _dinv_from(dp_ref)
    acc = accp_ref[0] + accp_ref[1]
    zp = acc * dinv + h_ref[...] * (dinv * dinv) + b_ref[...]
    t = _leaky(jnp.dot(zp, lw_ref[...], preferred_element_type=jnp.float32)
               + lb_ref[...])
    rows = lax.broadcasted_iota(jnp.int32, (RBLK, HH), 0) + i * RBLK
    t = jnp.where(rows < NN, t, 0.0)
    s1 = jnp.sum(t, axis=0, keepdims=True)
    s2 = jnp.sum(t * t, axis=0, keepdims=True)
    st = jnp.concatenate([s1, s2], axis=0)

    @pl.when(i == 0)
    def _init():
        stats_vmem[...] = st

    @pl.when(i > 0)
    def _accum():
        stats_vmem[...] = stats_vmem[...] + st

    @pl.when(i == NBLK - 1)
    def _head():
        s = stats_vmem[0]
        m = s / NN
        v = stats_vmem[1] / NN - m * m
        a = bng_ref[0] * lax.rsqrt(v + 1e-5)
        pooled = ((s / NN - m) * a + bnb_ref[0])[None, :]
        z = _leaky(jnp.dot(pooled, f1w_ref[...],
                           preferred_element_type=jnp.float32) + f1b_ref[...])
        z = _leaky(jnp.dot(z, f2w_ref[...],
                           preferred_element_type=jnp.float32) + f2b_ref[...])
        out_ref[...] = (jnp.dot(z, f3w_ref[...],
                                preferred_element_type=jnp.float32)
                        + f3b_ref[...])


def _mid_body(t_ref, stats_ref, bng_ref, bnb_ref, w1_ref, dp_ref,
              h1_ref, g1_ref):
    m = stats_ref[0] / NN
    v = stats_ref[1] / NN - m * m
    a = bng_ref[0] * lax.rsqrt(v + 1e-5)
    beta = bnb_ref[0] - m * a
    wp = a[:, None] * w1_ref[...]
    bp = jnp.dot(beta[None, :], w1_ref[...], preferred_element_type=jnp.float32)
    h1 = jnp.dot(t_ref[...], wp, preferred_element_type=jnp.float32) + bp
    h1_ref[...] = h1
    g1_ref[...] = h1 * _dinv_from(dp_ref)


def _row_spec(shape):
    return pl.BlockSpec(shape, lambda i: (i,) + (0,) * (len(shape) - 1))


def _full_spec(shape):
    n = len(shape)
    return pl.BlockSpec(shape, lambda i: (0,) * n)


_DP_SPEC = pl.BlockSpec((NCORE, RBLK, 1), lambda i: (0, i, 0))
_ACC_SPEC = pl.BlockSpec((NCORE, RBLK, HH), lambda i: (0, i, 0))
_ROW_SPEC = _row_spec((RBLK, HH))


def _tc_prep(dp3, xp, w0):
    return pl.pallas_call(
        _prep_body,
        grid=(NBLK,),
        in_specs=[_DP_SPEC, _ROW_SPEC, _full_spec((HH, HH))],
        out_specs=[_ROW_SPEC, _ROW_SPEC],
        out_shape=[jax.ShapeDtypeStruct((NPAD, HH), jnp.float32)] * 2,
    )(dp3, xp, w0)


def _tc_layer0(dp3, accp, h0, b, lw, lb):
    return pl.pallas_call(
        _layer0_body,
        grid=(NBLK,),
        in_specs=[_DP_SPEC, _ACC_SPEC, _ROW_SPEC, _full_spec((1, HH)),
                  _full_spec((HH, HH)), _full_spec((1, HH))],
        out_specs=[_ROW_SPEC, _full_spec((2, HH))],
        out_shape=[jax.ShapeDtypeStruct((NPAD, HH), jnp.float32),
                   jax.ShapeDtypeStruct((2, HH), jnp.float32)],
    )(dp3, accp, h0, b, lw, lb)


def _tc_mid(t, stats, bng, bnb, w1, dp3):
    return pl.pallas_call(
        _mid_body,
        grid=(NBLK,),
        in_specs=[_ROW_SPEC, _full_spec((2, HH)), _full_spec((1, HH)),
                  _full_spec((1, HH)), _full_spec((HH, HH)), _DP_SPEC],
        out_specs=[_ROW_SPEC, _ROW_SPEC],
        out_shape=[jax.ShapeDtypeStruct((NPAD, HH), jnp.float32)] * 2,
    )(t, stats, bng, bnb, w1, dp3)


def _tc_tail(dp3, accp, h1, b, lw, lb, bng, bnb, f1w, f1b, f2w, f2b, f3w, f3b):
    return pl.pallas_call(
        _tail_body,
        grid=(NBLK,),
        in_specs=[_DP_SPEC, _ACC_SPEC, _ROW_SPEC, _full_spec((1, HH)),
                  _full_spec((HH, HH)), _full_spec((1, HH)),
                  _full_spec((1, HH)), _full_spec((1, HH)),
                  _full_spec((HH, HH)), _full_spec((1, HH)),
                  _full_spec((HH, HH // 2)), _full_spec((1, HH // 2)),
                  _full_spec((HH // 2, 2)), _full_spec((1, 2))],
        out_specs=[_full_spec((1, 2))],
        out_shape=[jax.ShapeDtypeStruct((1, 2), jnp.float32)],
        scratch_shapes=[pltpu.VMEM((2, HH), jnp.float32)],
    )(dp3, accp, h1, b, lw, lb, bng, bnb, f1w, f1b, f2w, f2b, f3w, f3b)[0]


def kernel(x, edge_attr, W_gcn0, gcn0_b, lin0_W, lin0_b, bn0_g, bn0_b,
           W_gcn1, gcn1_b, lin1_W, lin1_b, bn1_g, bn1_b,
           fc1_W, fc1_b, fc2_W, fc2_b, fc3_W, fc3_b, edge_index, batch):
    row2 = edge_index[0].reshape(NWORK, EPW)
    col1 = edge_index[1]
    col3 = col1.reshape(NWORK, NCHUNK, CHUNK)
    ea1 = edge_attr.astype(jnp.float32)
    ea2 = ea1.reshape(NWORK, EPW)
    xp = jnp.pad(x.astype(jnp.float32), ((0, NPAD - NN), (0, 0)))
    zrow = jnp.zeros((RPT,), jnp.float32)
    ztile = jnp.zeros((RPT, HH), jnp.float32)

    degp = _sc_deg(col3, ea2, zrow)
    dp3 = degp.reshape(NCORE, NPAD, 1)

    h0, g0 = _tc_prep(dp3, xp, W_gcn0)
    acc0 = _sc_agg(row2, col1, ea1, g0, ztile)
    t, stats = _tc_layer0(dp3, acc0, h0, gcn0_b.reshape(1, HH),
                          lin0_W, lin0_b.reshape(1, HH))
    h1, g1 = _tc_mid(t, stats, bn0_g.reshape(1, HH), bn0_b.reshape(1, HH),
                     W_gcn1, dp3)
    acc1 = _sc_agg(row2, col1, ea1, g1, ztile)
    return _tc_tail(dp3, acc1, h1, gcn1_b.reshape(1, HH),
                    lin1_W, lin1_b.reshape(1, HH),
                    bn1_g.reshape(1, HH), bn1_b.reshape(1, HH),
                    fc1_W, fc1_b.reshape(1, HH), fc2_W,
                    fc2_b.reshape(1, HH // 2), fc3_W, fc3_b.reshape(1, 2))


# speedup vs baseline: 1.0962x; 1.0962x over previous
import functools

import jax
import jax.numpy as jnp
from jax import lax
from jax.experimental import pallas as pl
from jax.experimental.pallas import tpu as pltpu
from jax.experimental.pallas import tpu_sc as plsc

NN = 10000
EE = 320000
HH = 128
NPAD = 10240
NCORE = 2
NSUB = 16
NWORK = NCORE * NSUB
EPW = EE // NWORK
CHUNK = 80
NCHUNK = EPW // CHUNK
RPT = NPAD // NSUB
RBLK = 1024
NBLK = NPAD // RBLK

_mesh = plsc.VectorSubcoreMesh(core_axis_name="c", subcore_axis_name="s")


@functools.partial(
    pl.kernel,
    out_type=jax.ShapeDtypeStruct((NCORE, NPAD), jnp.float32),
    mesh=_mesh,
    scratch_types=[
        pltpu.VMEM((NCHUNK, CHUNK), jnp.int32),
        pltpu.VMEM((EPW,), jnp.float32),
        pltpu.VMEM_SHARED((NPAD,), jnp.float32),
        pltpu.SemaphoreType.DMA,
    ],
)
def _sc_deg(col_hbm, ea_hbm, zrow_hbm, deg_hbm, col_v, w_v, acc_sh, dsem):
    cid = lax.axis_index("c")
    sid = lax.axis_index("s")
    wid = cid * NSUB + sid
    pltpu.sync_copy(col_hbm.at[wid], col_v)
    pltpu.sync_copy(ea_hbm.at[wid], w_v)

    def _abs_body(i, carry):
        w_v[pl.ds(i * 16, 16)] = jnp.abs(w_v[pl.ds(i * 16, 16)])
        return carry

    lax.fori_loop(0, EPW // 16, _abs_body, 0)
    pltpu.sync_copy(zrow_hbm, acc_sh.at[pl.ds(sid * RPT, RPT)])
    plsc.subcore_barrier()

    fire = 25

    def _round_body(r, carry):
        for j in range(fire):
            pltpu.async_copy(
                w_v.at[pl.ds((r * fire + j) * CHUNK, CHUNK)],
                acc_sh.at[col_v.at[r * fire + j]],
                dsem,
                add=True,
            )
        for j in range(fire):
            pltpu.make_async_copy(
                w_v.at[pl.ds(j * CHUNK, CHUNK)],
                acc_sh.at[col_v.at[j]],
                dsem,
            ).wait()
        return carry

    lax.fori_loop(0, NCHUNK // fire, _round_body, 0)
    plsc.subcore_barrier()
    pltpu.sync_copy(acc_sh.at[pl.ds(sid * RPT, RPT)],
                    deg_hbm.at[cid, pl.ds(sid * RPT, RPT)])


@functools.partial(
    pl.kernel,
    out_type=jax.ShapeDtypeStruct((NCORE, NPAD, HH), jnp.float32),
    mesh=_mesh,
    scratch_types=[
        pltpu.VMEM((EPW,), jnp.int32),
        pltpu.VMEM((NCHUNK, CHUNK), jnp.int32),
        pltpu.VMEM((CHUNK,), jnp.float32),
        pltpu.VMEM((CHUNK,), jnp.float32),
        pltpu.VMEM((CHUNK, HH), jnp.float32),
        pltpu.VMEM((CHUNK, HH), jnp.float32),
        pltpu.VMEM_SHARED((NPAD, HH), jnp.float32),
        pltpu.SemaphoreType.DMA,
        pltpu.SemaphoreType.DMA,
    ],
)
def _sc_agg(row_hbm, col_hbm, ea_hbm, g_hbm, ztile_hbm, acc_hbm,
            row_v, col_v, ea_a, ea_b, msg_a, msg_b, acc_sh, ga, gb):
    cid = lax.axis_index("c")
    sid = lax.axis_index("s")
    wid = cid * NSUB + sid
    pltpu.sync_copy(row_hbm.at[wid], row_v)
    pltpu.sync_copy(col_hbm.at[wid], col_v)
    pltpu.sync_copy(ztile_hbm, acc_sh.at[pl.ds(sid * RPT, RPT)])
    plsc.subcore_barrier()

    def _gather_start(k, buf, eabuf, sem):
        pltpu.async_copy(g_hbm.at[row_v.at[pl.ds(k * CHUNK, CHUNK)]],
                         buf, sem)
        pltpu.async_copy(ea_hbm.at[pl.ds(wid * EPW + k * CHUNK, CHUNK)],
                         eabuf, sem)

    def _gather_wait(buf, eabuf, sem):
        pltpu.make_async_copy(g_hbm.at[row_v.at[pl.ds(0, CHUNK)]],
                              buf, sem).wait()
        pltpu.make_async_copy(ea_hbm.at[pl.ds(0, CHUNK)],
                              eabuf, sem).wait()

    def _scatter(k, buf):
        pltpu.sync_copy(buf, acc_sh.at[col_v.at[k]], add=True)

    def _scale(buf, eabuf):
        for j in range(CHUNK // 16):
            wvec = jnp.abs(eabuf[pl.ds(j * 16, 16)])
            for l in range(16):
                e = j * 16 + l
                wv = jnp.full((16,), wvec[l], jnp.float32)
                for cg in range(HH // 16):
                    buf[e, pl.ds(cg * 16, 16)] = (
                        buf[e, pl.ds(cg * 16, 16)] * wv)

    def _phase(k, cur, ea_cur, curg, oth, ea_oth, othg):
        _gather_wait(cur, ea_cur, curg)

        @pl.when(k + 1 < NCHUNK)
        def _prefetch():
            _gather_start(k + 1, oth, ea_oth, othg)

        _scale(cur, ea_cur)
        _scatter(k, cur)

    _gather_start(0, msg_a, ea_a, ga)

    def _chunk_body(k, carry):
        @pl.when(k % 2 == 0)
        def _even():
            _phase(k, msg_a, ea_a, ga, msg_b, ea_b, gb)

        @pl.when(k % 2 == 1)
        def _odd():
            _phase(k, msg_b, ea_b, gb, msg_a, ea_a, ga)

        return carry

    lax.fori_loop(0, NCHUNK, _chunk_body, 0)
    plsc.subcore_barrier()
    pltpu.sync_copy(acc_sh.at[pl.ds(sid * RPT, RPT)],
                    acc_hbm.at[cid, pl.ds(sid * RPT, RPT)])


def _leaky(x):
    return jnp.where(x > 0, x, 0.2 * x)


def _dinv_from(dp_ref):
    deg = dp_ref[0] + dp_ref[1] + 1.0
    return jnp.where(deg > 0, lax.rsqrt(deg), 0.0)


def _prep_body(dp_ref, x_ref, w_ref, h_ref, g_ref):
    dinv = _dinv_from(dp_ref)
    h = jnp.dot(x_ref[...], w_ref[...], preferred_element_type=jnp.float32)
    h_ref[...] = h
    g_ref[...] = h * dinv


def _block_layer_body(dp_ref, accp_ref, h_ref, b_ref, lw_ref, lb_ref,
                      t_ref, stats_ref, *, mask_rows):
    i = pl.program_id(0)
    dinv = _dinv_from(dp_ref)
    acc = accp_ref[0] + accp_ref[1]
    zp = acc * dinv + h_ref[...] * (dinv * dinv) + b_ref[...]
    t = _leaky(jnp.dot(zp, lw_ref[...], preferred_element_type=jnp.float32)
               + lb_ref[...])
    if mask_rows:
        rows = lax.broadcasted_iota(jnp.int32, (RBLK, HH), 0) + i * RBLK
        t = jnp.where(rows < NN, t, 0.0)
    if t_ref is not None:
        t_ref[...] = t
    s1 = jnp.sum(t, axis=0, keepdims=True)
    s2 = jnp.sum(t * t, axis=0, keepdims=True)
    st = jnp.concatenate([s1, s2], axis=0)

    @pl.when(i == 0)
    def _init():
        stats_ref[...] = st

    @pl.when(i > 0)
    def _accum():
        stats_ref[...] = stats_ref[...] + st


def _layer0_body(dp_ref, accp_ref, h_ref, b_ref, lw_ref, lb_ref,
                 t_ref, stats_ref):
    _block_layer_body(dp_ref, accp_ref, h_ref, b_ref, lw_ref, lb_ref,
                      t_ref, stats_ref, mask_rows=True)


def _tail_body(dp_ref, accp_ref, h_ref, b_ref, lw_ref, lb_ref,
               bng_ref, bnb_ref, f1w_ref, f1b_ref, f2w_ref, f2b_ref,
               f3w_ref, f3b_ref, out_ref, stats_vmem):
    i = pl.program_id(0)
    dinv = _dinv_from(dp_ref)
    acc = accp_ref[0] + accp_ref[1]
    zp = acc * dinv + h_ref[...] * (dinv * dinv) + b_ref[...]
    t = _leaky(jnp.dot(zp, lw_ref[...], preferred_element_type=jnp.float32)
               + lb_ref[...])
    rows = lax.broadcasted_iota(jnp.int32, (RBLK, HH), 0) + i * RBLK
    t = jnp.where(rows < NN, t, 0.0)
    s1 = jnp.sum(t, axis=0, keepdims=True)
    s2 = jnp.sum(t * t, axis=0, keepdims=True)
    st = jnp.concatenate([s1, s2], axis=0)

    @pl.when(i == 0)
    def _init():
        stats_vmem[...] = st

    @pl.when(i > 0)
    def _accum():
        stats_vmem[...] = stats_vmem[...] + st

    @pl.when(i == NBLK - 1)
    def _head():
        s = stats_vmem[0]
        m = s / NN
        v = stats_vmem[1] / NN - m * m
        a = bng_ref[0] * lax.rsqrt(v + 1e-5)
        pooled = ((s / NN - m) * a + bnb_ref[0])[None, :]
        z = _leaky(jnp.dot(pooled, f1w_ref[...],
                           preferred_element_type=jnp.float32) + f1b_ref[...])
        z = _leaky(jnp.dot(z, f2w_ref[...],
                           preferred_element_type=jnp.float32) + f2b_ref[...])
        out_ref[...] = (jnp.dot(z, f3w_ref[...],
                                preferred_element_type=jnp.float32)
                        + f3b_ref[...])


def _mid_body(t_ref, stats_ref, bng_ref, bnb_ref, w1_ref, dp_ref,
              h1_ref, g1_ref):
    m = stats_ref[0] / NN
    v = stats_ref[1] / NN - m * m
    a = bng_ref[0] * lax.rsqrt(v + 1e-5)
    beta = bnb_ref[0] - m * a
    wp = a[:, None] * w1_ref[...]
    bp = jnp.dot(beta[None, :], w1_ref[...], preferred_element_type=jnp.float32)
    h1 = jnp.dot(t_ref[...], wp, preferred_element_type=jnp.float32) + bp
    h1_ref[...] = h1
    g1_ref[...] = h1 * _dinv_from(dp_ref)


def _row_spec(shape):
    return pl.BlockSpec(shape, lambda i: (i,) + (0,) * (len(shape) - 1))


def _full_spec(shape):
    n = len(shape)
    return pl.BlockSpec(shape, lambda i: (0,) * n)


_DP_SPEC = pl.BlockSpec((NCORE, RBLK, 1), lambda i: (0, i, 0))
_ACC_SPEC = pl.BlockSpec((NCORE, RBLK, HH), lambda i: (0, i, 0))
_ROW_SPEC = _row_spec((RBLK, HH))


def _tc_prep(dp3, xp, w0):
    return pl.pallas_call(
        _prep_body,
        grid=(NBLK,),
        in_specs=[_DP_SPEC, _ROW_SPEC, _full_spec((HH, HH))],
        out_specs=[_ROW_SPEC, _ROW_SPEC],
        out_shape=[jax.ShapeDtypeStruct((NPAD, HH), jnp.float32)] * 2,
    )(dp3, xp, w0)


def _tc_layer0(dp3, accp, h0, b, lw, lb):
    return pl.pallas_call(
        _layer0_body,
        grid=(NBLK,),
        in_specs=[_DP_SPEC, _ACC_SPEC, _ROW_SPEC, _full_spec((1, HH)),
                  _full_spec((HH, HH)), _full_spec((1, HH))],
        out_specs=[_ROW_SPEC, _full_spec((2, HH))],
        out_shape=[jax.ShapeDtypeStruct((NPAD, HH), jnp.float32),
                   jax.ShapeDtypeStruct((2, HH), jnp.float32)],
    )(dp3, accp, h0, b, lw, lb)


def _tc_mid(t, stats, bng, bnb, w1, dp3):
    return pl.pallas_call(
        _mid_body,
        grid=(NBLK,),
        in_specs=[_ROW_SPEC, _full_spec((2, HH)), _full_spec((1, HH)),
                  _full_spec((1, HH)), _full_spec((HH, HH)), _DP_SPEC],
        out_specs=[_ROW_SPEC, _ROW_SPEC],
        out_shape=[jax.ShapeDtypeStruct((NPAD, HH), jnp.float32)] * 2,
    )(t, stats, bng, bnb, w1, dp3)


def _tc_tail(dp3, accp, h1, b, lw, lb, bng, bnb, f1w, f1b, f2w, f2b, f3w, f3b):
    return pl.pallas_call(
        _tail_body,
        grid=(NBLK,),
        in_specs=[_DP_SPEC, _ACC_SPEC, _ROW_SPEC, _full_spec((1, HH)),
                  _full_spec((HH, HH)), _full_spec((1, HH)),
                  _full_spec((1, HH)), _full_spec((1, HH)),
                  _full_spec((HH, HH)), _full_spec((1, HH)),
                  _full_spec((HH, HH // 2)), _full_spec((1, HH // 2)),
                  _full_spec((HH // 2, 2)), _full_spec((1, 2))],
        out_specs=[_full_spec((1, 2))],
        out_shape=[jax.ShapeDtypeStruct((1, 2), jnp.float32)],
        scratch_shapes=[pltpu.VMEM((2, HH), jnp.float32)],
    )(dp3, accp, h1, b, lw, lb, bng, bnb, f1w, f1b, f2w, f2b, f3w, f3b)[0]


def kernel(x, edge_attr, W_gcn0, gcn0_b, lin0_W, lin0_b, bn0_g, bn0_b,
           W_gcn1, gcn1_b, lin1_W, lin1_b, bn1_g, bn1_b,
           fc1_W, fc1_b, fc2_W, fc2_b, fc3_W, fc3_b, edge_index, batch):
    row2 = edge_index[0].reshape(NWORK, EPW)
    col1 = edge_index[1]
    col3 = col1.reshape(NWORK, NCHUNK, CHUNK)
    ea1 = edge_attr.astype(jnp.float32)
    ea2 = ea1.reshape(NWORK, EPW)
    xp = jnp.pad(x.astype(jnp.float32), ((0, NPAD - NN), (0, 0)))
    zrow = jnp.zeros((RPT,), jnp.float32)
    ztile = jnp.zeros((RPT, HH), jnp.float32)

    degp = _sc_deg(col3, ea2, zrow)
    dp3 = degp.reshape(NCORE, NPAD, 1)

    h0, g0 = _tc_prep(dp3, xp, W_gcn0)
    acc0 = _sc_agg(row2, col3, ea1, g0, ztile)
    t, stats = _tc_layer0(dp3, acc0, h0, gcn0_b.reshape(1, HH),
                          lin0_W, lin0_b.reshape(1, HH))
    h1, g1 = _tc_mid(t, stats, bn0_g.reshape(1, HH), bn0_b.reshape(1, HH),
                     W_gcn1, dp3)
    acc1 = _sc_agg(row2, col3, ea1, g1, ztile)
    return _tc_tail(dp3, acc1, h1, gcn1_b.reshape(1, HH),
                    lin1_W, lin1_b.reshape(1, HH),
                    bn1_g.reshape(1, HH), bn1_b.reshape(1, HH),
                    fc1_W, fc1_b.reshape(1, HH), fc2_W,
                    fc2_b.reshape(1, HH // 2), fc3_W, fc3_b.reshape(1, 2))
